# baseline (speedup 1.0000x reference)
import functools

import jax
import jax.numpy as jnp
from jax import lax
from jax.experimental import pallas as pl
from jax.experimental.pallas import tpu as pltpu
from jax.experimental.pallas import tpu_sc as plsc

RES_H = 16
RES_W = 16
FEAT = 4
GRID_D = RES_H * RES_W * FEAT
OUT_D = 4
EX_PER_BLK = 8


def _sc_gather(table, idx):
    num_rows, d = table.shape
    b = idx.shape[0]
    info = plsc.get_sparse_core_info()
    nw = info.num_cores * info.num_subcores
    b_per_w = b // nw
    chunk = 64
    n_chunks = b_per_w // chunk
    mesh = plsc.VectorSubcoreMesh(core_axis_name="c", subcore_axis_name="s")

    @functools.partial(
        pl.kernel,
        out_type=jax.ShapeDtypeStruct((b, d), jnp.float32),
        mesh=mesh,
        scratch_types=[
            pltpu.VMEM((chunk,), jnp.int32),
            pltpu.VMEM((chunk, d), jnp.float32),
            pltpu.SemaphoreType.DMA,
        ],
    )
    def gather_kernel(table_hbm, idx_hbm, out_hbm, idx_v, rows_v, sem):
        wid = lax.axis_index("s") * info.num_cores + lax.axis_index("c")
        base = wid * b_per_w
        for c in range(n_chunks):
            off = base + c * chunk
            pltpu.sync_copy(idx_hbm.at[pl.ds(off, chunk)], idx_v)
            pltpu.async_copy(table_hbm.at[idx_v], rows_v, sem).wait()
            pltpu.sync_copy(rows_v, out_hbm.at[pl.ds(off, chunk)])

    return gather_kernel(table, idx)


def _interp_mlp_body(xx_ref, yy_ref, g_ref, w0_ref, b0_ref, w1_ref, b1_ref,
                     w2_ref, b2_ref, o_ref):
    n = EX_PER_BLK * 128
    ky = EX_PER_BLK * RES_H

    xx = xx_ref[...]
    yy = yy_ref[...]
    lx = (xx + 0.5) * (RES_W - 1.0)
    ly = (yy + 0.5) * (RES_H - 1.0)
    x0f = jnp.floor(lx)
    y0f = jnp.floor(ly)
    wx = lx - x0f
    wy = ly - y0f
    x0 = jnp.clip(x0f.astype(jnp.int32), 0, RES_W - 1)
    x1 = jnp.minimum(x0 + 1, RES_W - 1)
    y0 = jnp.clip(y0f.astype(jnp.int32), 0, RES_H - 1)
    y1 = jnp.minimum(y0 + 1, RES_H - 1)

    q = lax.broadcasted_iota(jnp.int32, (n, ky), 1)
    r = lax.broadcasted_iota(jnp.int32, (n, ky), 0)
    same_ex = (q >> 4) == (r >> 7)
    yq = q & (RES_H - 1)
    ymat = (jnp.where(same_ex & (yq == y0), 1.0 - wy, 0.0)
            + jnp.where(same_ex & (yq == y1), wy, 0.0))

    g = g_ref[...]
    tmp = jnp.dot(ymat, g, preferred_element_type=jnp.float32)

    j = lax.broadcasted_iota(jnp.int32, (n, RES_W * FEAT), 1)
    xj = j >> 2
    xmat = (jnp.where(xj == x0, 1.0 - wx, 0.0)
            + jnp.where(xj == x1, wx, 0.0))
    prod = tmp * xmat

    jj = lax.broadcasted_iota(jnp.int32, (RES_W * FEAT, FEAT), 0)
    cc = lax.broadcasted_iota(jnp.int32, (RES_W * FEAT, FEAT), 1)
    sel = ((jj & 3) == cc).astype(jnp.float32)
    sw0 = jnp.dot(sel, w0_ref[...], preferred_element_type=jnp.float32)

    h = jnp.dot(prod, sw0, preferred_element_type=jnp.float32) + b0_ref[...]
    h = jnp.where(h >= 0.0, h, 0.01 * h)
    h = jnp.dot(h, w1_ref[...], preferred_element_type=jnp.float32) + b1_ref[...]
    h = jnp.where(h >= 0.0, h, 0.01 * h)
    o_ref[...] = (jnp.dot(h, w2_ref[...], preferred_element_type=jnp.float32)
                  + b2_ref[...])


def kernel(idx, x, emb, w0, b0, w1, b1, w2, b2):
    b = idx.shape[0]
    p = x.shape[1]
    n_pts = b * p

    g = _sc_gather(emb, idx.astype(jnp.int32))
    gs = jnp.reshape(g, (b * RES_H, RES_W * FEAT))
    xx = jnp.reshape(x[..., 0], (n_pts, 1))
    yy = jnp.reshape(x[..., 1], (n_pts, 1))

    blk_pts = EX_PER_BLK * p
    grid = (b // EX_PER_BLK,)
    full = lambda i: (0, 0)
    out = pl.pallas_call(
        _interp_mlp_body,
        grid=grid,
        in_specs=[
            pl.BlockSpec((blk_pts, 1), lambda i: (i, 0)),
            pl.BlockSpec((blk_pts, 1), lambda i: (i, 0)),
            pl.BlockSpec((EX_PER_BLK * RES_H, RES_W * FEAT), lambda i: (i, 0)),
            pl.BlockSpec((FEAT, 64), full),
            pl.BlockSpec((1, 64), full),
            pl.BlockSpec((64, 64), full),
            pl.BlockSpec((1, 64), full),
            pl.BlockSpec((64, OUT_D), full),
            pl.BlockSpec((1, OUT_D), full),
        ],
        out_specs=pl.BlockSpec((blk_pts, OUT_D), lambda i: (i, 0)),
        out_shape=jax.ShapeDtypeStruct((n_pts, OUT_D), jnp.float32),
        compiler_params=pltpu.CompilerParams(
            dimension_semantics=("arbitrary",)),
    )(xx, yy, gs, w0, jnp.reshape(b0, (1, 64)), w1, jnp.reshape(b1, (1, 64)),
      w2, jnp.reshape(b2, (1, OUT_D)))
    return jnp.reshape(out, (b, p, OUT_D))


# speedup vs baseline: 1.9239x; 1.9239x over previous
import functools

import jax
import jax.numpy as jnp
from jax import lax
from jax.experimental import pallas as pl
from jax.experimental.pallas import tpu as pltpu
from jax.experimental.pallas import tpu_sc as plsc

RES_H = 16
RES_W = 16
FEAT = 4
GRID_D = RES_H * RES_W * FEAT
OUT_D = 4
EX_PER_BLK = 8
NPTS = 128

_DN0 = (((0,), (0,)), ((), ()))


def _sc_gather(table, idx):
    num_rows, d = table.shape
    b = idx.shape[0]
    info = plsc.get_sparse_core_info()
    nw = info.num_cores * info.num_subcores
    b_per_w = b // nw
    chunk = 64
    n_chunks = b_per_w // chunk
    mesh = plsc.VectorSubcoreMesh(core_axis_name="c", subcore_axis_name="s")

    @functools.partial(
        pl.kernel,
        out_type=jax.ShapeDtypeStruct((b, d), jnp.float32),
        mesh=mesh,
        scratch_types=[
            pltpu.VMEM((chunk,), jnp.int32),
            pltpu.VMEM((chunk, d), jnp.float32),
            pltpu.SemaphoreType.DMA,
        ],
    )
    def gather_kernel(table_hbm, idx_hbm, out_hbm, idx_v, rows_v, sem):
        wid = lax.axis_index("s") * info.num_cores + lax.axis_index("c")
        base = wid * b_per_w
        for c in range(n_chunks):
            off = base + c * chunk
            pltpu.sync_copy(idx_hbm.at[pl.ds(off, chunk)], idx_v)
            pltpu.async_copy(table_hbm.at[idx_v], rows_v, sem).wait()
            pltpu.sync_copy(rows_v, out_hbm.at[pl.ds(off, chunk)])

    return gather_kernel(table, idx)


def _interp_mlp_body(xt_ref, g_ref, ycol_ref, xcol_ref, sel_ref,
                     w0_ref, b0_ref, w1_ref, b1_ref, w2_ref, b2_ref, o_ref):
    xt = xt_ref[...]
    gsb = g_ref[...]
    ycol = ycol_ref[...]
    xcol = xcol_ref[...]
    sel = sel_ref[...]

    feat_parts = []
    for e in range(EX_PER_BLK):
        lx = (xt[0, e] + 0.5) * (RES_W - 1.0)
        ly = (xt[1, e] + 0.5) * (RES_H - 1.0)
        yhat = jnp.maximum(0.0, 1.0 - jnp.abs(ly - ycol))
        xhat = jnp.maximum(0.0, 1.0 - jnp.abs(lx - xcol))
        g_e = gsb[e * RES_H:(e + 1) * RES_H, :]
        tmp = lax.dot_general(g_e, yhat, _DN0,
                              preferred_element_type=jnp.float32)
        prod = tmp * xhat
        feat_parts.append(
            lax.dot_general(sel, prod, _DN0,
                            preferred_element_type=jnp.float32))
    featt = jnp.concatenate(feat_parts, axis=1)

    h = lax.dot_general(w0_ref[...], featt, _DN0,
                        preferred_element_type=jnp.float32) + b0_ref[...]
    h = jnp.maximum(h, 0.01 * h)
    h = lax.dot_general(w1_ref[...], h, _DN0,
                        preferred_element_type=jnp.float32) + b1_ref[...]
    h = jnp.maximum(h, 0.01 * h)
    o_ref[...] = (lax.dot_general(h, w2_ref[...], _DN0,
                                  preferred_element_type=jnp.float32)
                  + b2_ref[...])


def kernel(idx, x, emb, w0, b0, w1, b1, w2, b2):
    b = idx.shape[0]
    p = x.shape[1]
    n_pts = b * p

    g = _sc_gather(emb, idx.astype(jnp.int32))
    gs = jnp.reshape(g, (b * RES_H, RES_W * FEAT))
    xt = jnp.transpose(x, (2, 0, 1))

    ycol = jnp.arange(RES_H, dtype=jnp.float32).reshape(RES_H, 1)
    xcol = (jnp.arange(RES_W * FEAT, dtype=jnp.int32) >> 2).astype(
        jnp.float32).reshape(RES_W * FEAT, 1)
    sel = ((jnp.arange(RES_W * FEAT, dtype=jnp.int32)[:, None] & 3)
           == jnp.arange(FEAT, dtype=jnp.int32)[None, :]).astype(jnp.float32)

    blk_pts = EX_PER_BLK * p
    full = lambda i: (0, 0)
    out = pl.pallas_call(
        _interp_mlp_body,
        grid=(b // EX_PER_BLK,),
        in_specs=[
            pl.BlockSpec((2, EX_PER_BLK, p), lambda i: (0, i, 0)),
            pl.BlockSpec((EX_PER_BLK * RES_H, RES_W * FEAT), lambda i: (i, 0)),
            pl.BlockSpec((RES_H, 1), full),
            pl.BlockSpec((RES_W * FEAT, 1), full),
            pl.BlockSpec((RES_W * FEAT, FEAT), full),
            pl.BlockSpec((FEAT, 64), full),
            pl.BlockSpec((64, 1), full),
            pl.BlockSpec((64, 64), full),
            pl.BlockSpec((64, 1), full),
            pl.BlockSpec((64, OUT_D), full),
            pl.BlockSpec((1, OUT_D), full),
        ],
        out_specs=pl.BlockSpec((blk_pts, OUT_D), lambda i: (i, 0)),
        out_shape=jax.ShapeDtypeStruct((n_pts, OUT_D), jnp.float32),
        compiler_params=pltpu.CompilerParams(
            dimension_semantics=("arbitrary",)),
    )(xt, gs, ycol, xcol, sel, w0, jnp.reshape(b0, (64, 1)), w1,
      jnp.reshape(b1, (64, 1)), w2, jnp.reshape(b2, (1, OUT_D)))
    return jnp.reshape(out, (b, p, OUT_D))


# speedup vs baseline: 3.1327x; 1.6283x over previous
import functools

import jax
import jax.numpy as jnp
from jax import lax
from jax.experimental import pallas as pl
from jax.experimental.pallas import tpu as pltpu
from jax.experimental.pallas import tpu_sc as plsc

RES_H = 16
RES_W = 16
FEAT = 4
GRID_D = RES_H * RES_W * FEAT
OUT_D = 4
EX_PER_BLK = 32
NPTS = 128

_DN0 = (((0,), (0,)), ((), ()))


def _sc_gather(table, idx):
    num_rows, d = table.shape
    b = idx.shape[0]
    info = plsc.get_sparse_core_info()
    nw = info.num_cores * info.num_subcores
    b_per_w = b // nw
    chunk = 64
    n_chunks = b_per_w // chunk
    mesh = plsc.VectorSubcoreMesh(core_axis_name="c", subcore_axis_name="s")

    @functools.partial(
        pl.kernel,
        out_type=jax.ShapeDtypeStruct((b, d), jnp.float32),
        mesh=mesh,
        scratch_types=[
            pltpu.VMEM((chunk,), jnp.int32),
            pltpu.VMEM((chunk, d), jnp.float32),
            pltpu.SemaphoreType.DMA,
        ],
    )
    def gather_kernel(table_hbm, idx_hbm, out_hbm, idx_v, rows_v, sem):
        wid = lax.axis_index("s") * info.num_cores + lax.axis_index("c")
        base = wid * b_per_w
        for c in range(n_chunks):
            off = base + c * chunk
            pltpu.sync_copy(idx_hbm.at[pl.ds(off, chunk)], idx_v)
            pltpu.async_copy(table_hbm.at[idx_v], rows_v, sem).wait()
            pltpu.sync_copy(rows_v, out_hbm.at[pl.ds(off, chunk)])

    return gather_kernel(table, idx)


def _interp_mlp_body(xr_ref, g_ref, dmat_ref, ycol_ref, xcol_ref, selt_ref,
                     w0t_ref, b0_ref, w1t_ref, b1_ref, w2t_ref, b2_ref, o_ref):
    xr = xr_ref[...]
    gsb = g_ref[...]
    ycol = ycol_ref[...]
    xcol = xcol_ref[...]
    selt = selt_ref[...]

    loc = jnp.dot(xr, dmat_ref[...],
                  preferred_element_type=jnp.float32) + 7.5

    feat_parts = []
    for e in range(EX_PER_BLK):
        lx = loc[e, :NPTS]
        ly = loc[e, NPTS:]
        yhat = jnp.maximum(0.0, 1.0 - jnp.abs(ly - ycol))
        xhat = jnp.maximum(0.0, 1.0 - jnp.abs(lx - xcol))
        g_e = gsb[e * RES_H:(e + 1) * RES_H, :]
        tmp = lax.dot_general(g_e, yhat, _DN0,
                              preferred_element_type=jnp.float32)
        prod = tmp * xhat
        feat_parts.append(
            jnp.dot(selt, prod, preferred_element_type=jnp.float32))
    featt = jnp.concatenate(feat_parts, axis=1)

    h = jnp.dot(w0t_ref[...], featt,
                preferred_element_type=jnp.float32) + b0_ref[...]
    h = jnp.maximum(h, 0.01 * h)
    h = jnp.dot(w1t_ref[...], h,
                preferred_element_type=jnp.float32) + b1_ref[...]
    h = jnp.maximum(h, 0.01 * h)
    ot = jnp.dot(w2t_ref[...], h,
                 preferred_element_type=jnp.float32) + b2_ref[...]
    o_ref[...] = jnp.transpose(ot)


def kernel(idx, x, emb, w0, b0, w1, b1, w2, b2):
    b = idx.shape[0]
    p = x.shape[1]
    n_pts = b * p

    g = _sc_gather(emb, idx.astype(jnp.int32))
    gs = jnp.reshape(g, (b * RES_H, RES_W * FEAT))
    xr = jnp.reshape(x, (b, 2 * p))

    i2 = jnp.arange(2 * p, dtype=jnp.int32)
    j2 = jnp.arange(2 * p, dtype=jnp.int32)
    dmat = jnp.where((((i2[:, None] & 1) * p + (i2[:, None] >> 1)) == j2[None, :]),
                     jnp.float32(RES_W - 1), jnp.float32(0.0))

    ycol = jnp.arange(RES_H, dtype=jnp.float32).reshape(RES_H, 1)
    xcol = (jnp.arange(RES_W * FEAT, dtype=jnp.int32) >> 2).astype(
        jnp.float32).reshape(RES_W * FEAT, 1)
    selt = ((jnp.arange(RES_W * FEAT, dtype=jnp.int32)[None, :] & 3)
            == jnp.arange(FEAT, dtype=jnp.int32)[:, None]).astype(jnp.float32)

    blk_pts = EX_PER_BLK * p
    full = lambda i: (0, 0)
    out = pl.pallas_call(
        _interp_mlp_body,
        grid=(b // EX_PER_BLK,),
        in_specs=[
            pl.BlockSpec((EX_PER_BLK, 2 * p), lambda i: (i, 0)),
            pl.BlockSpec((EX_PER_BLK * RES_H, RES_W * FEAT), lambda i: (i, 0)),
            pl.BlockSpec((2 * p, 2 * p), full),
            pl.BlockSpec((RES_H, 1), full),
            pl.BlockSpec((RES_W * FEAT, 1), full),
            pl.BlockSpec((FEAT, RES_W * FEAT), full),
            pl.BlockSpec((64, FEAT), full),
            pl.BlockSpec((64, 1), full),
            pl.BlockSpec((64, 64), full),
            pl.BlockSpec((64, 1), full),
            pl.BlockSpec((OUT_D, 64), full),
            pl.BlockSpec((OUT_D, 1), full),
        ],
        out_specs=pl.BlockSpec((blk_pts, OUT_D), lambda i: (i, 0)),
        out_shape=jax.ShapeDtypeStruct((n_pts, OUT_D), jnp.float32),
        compiler_params=pltpu.CompilerParams(
            dimension_semantics=("arbitrary",)),
    )(xr, gs, dmat, ycol, xcol, selt, jnp.transpose(w0),
      jnp.reshape(b0, (64, 1)), jnp.transpose(w1), jnp.reshape(b1, (64, 1)),
      jnp.transpose(w2), jnp.reshape(b2, (OUT_D, 1)))
    return jnp.reshape(out, (b, p, OUT_D))
